# Initial kernel scaffold; baseline (speedup 1.0000x reference)
import jax
import jax.numpy as jnp
from jax.experimental import pallas as pl


def kernel(s_node_ids, s_edge_index, s_edge_attr, s_batch, depth, id_W1, id_b1, id_W2, id_b2, e_W1, e_b1, e_W2, e_b2, c1_W1, c1_b1, c1_W2, c1_b2, c2_W1, c2_b1, c2_W2, c2_b2, r_W1, r_b1, r_W2, r_b2):
    raise NotImplementedError("write your pallas kernel here")


# speedup vs baseline: 3.0231x; 3.0231x over previous
import functools

import jax
import jax.numpy as jnp
from jax import lax
from jax.experimental import pallas as pl
from jax.experimental.pallas import tpu as pltpu
from jax.experimental.pallas import tpu_sc as plsc

_TWO48 = float(2 ** 48 - 1)
_N = 10000
_E = 320000
_H = 128
_B = 64
_NC = 2
_NS = 16
_NW = _NC * _NS
_EPW = _E // _NW
_CHUNK = 80
_NCHUNK = _EPW // _CHUNK
_NP = 10240
_RPS = _NP // _NS
_HV = _H // 16


def _node_mlp_body(ids_ref, w1_ref, b1_ref, w2_ref, b2_ref, out_ref):
    x0 = jnp.clip(ids_ref[...].astype(jnp.float32) / _TWO48, 0.0, 1.0)
    h = jax.nn.relu(x0 * w1_ref[...] + b1_ref[...])
    out_ref[...] = (
        jnp.dot(h, w2_ref[...], preferred_element_type=jnp.float32)
        + b2_ref[...]
    )


def _edge_mlp_body(a_ref, w1_ref, b1_ref, w2_ref, b2_ref, out_ref):
    h = jax.nn.relu(
        jnp.dot(a_ref[...], w1_ref[...], preferred_element_type=jnp.float32)
        + b1_ref[...]
    )
    out_ref[...] = (
        jnp.dot(h, w2_ref[...], preferred_element_type=jnp.float32)
        + b2_ref[...]
    )


def _update_body(x_ref, agg_ref, w1_ref, b1_ref, w2_ref, b2_ref, out_ref):
    z = x_ref[...] + agg_ref[0] + agg_ref[1]
    h = jax.nn.relu(
        jnp.dot(z, w1_ref[...], preferred_element_type=jnp.float32)
        + b1_ref[...]
    )
    out_ref[...] = jax.nn.relu(
        jnp.dot(h, w2_ref[...], preferred_element_type=jnp.float32)
        + b2_ref[...]
    )


def _pool_head_body(h_ref, batch_ref, depth_ref, w1a_ref, w1b_ref, b1_ref,
                    w2_ref, b2_ref, out_ref):
    h = h_ref[...]
    bt = batch_ref[...]
    gids = lax.broadcasted_iota(jnp.int32, (_N, _B), 1)
    onehot = (bt == gids).astype(jnp.float32)
    sums = lax.dot_general(
        onehot, h, (((0,), (0,)), ((), ())),
        preferred_element_type=jnp.float32)
    cnts = jnp.sum(onehot, axis=0)[:, None]
    rep = sums / jnp.maximum(cnts, 1.0)
    r1 = jax.nn.relu(
        jnp.dot(rep, w1a_ref[...], preferred_element_type=jnp.float32)
        + depth_ref[...] * w1b_ref[...]
        + b1_ref[...])
    out_ref[...] = (
        jnp.dot(r1, w2_ref[...], preferred_element_type=jnp.float32)
        + b2_ref[...]
    )


def _mp_body(x_hbm, e_hbm, src_hbm, dst_hbm, zeros_hbm, out_hbm,
             idx_v, dst_v, xr_v, ev_v, aggr, sem):
    c = lax.axis_index("c")
    s = lax.axis_index("s")
    wid = s * _NC + c
    base = wid * _EPW

    pltpu.sync_copy(zeros_hbm.at[pl.ds(s * _RPS, _RPS)],
                    aggr.at[pl.ds(s * _RPS, _RPS)])
    plsc.subcore_barrier()

    def chunk_body(k, carry):
        off = base + k * _CHUNK
        pltpu.sync_copy(src_hbm.at[pl.ds(off, _CHUNK)], idx_v)
        pltpu.sync_copy(dst_hbm.at[pl.ds(off, _CHUNK)], dst_v)
        gcp = pltpu.async_copy(x_hbm.at[idx_v], xr_v, sem)
        pltpu.sync_copy(e_hbm.at[pl.ds(off, _CHUNK)], ev_v)
        gcp.wait()

        def row_body(r, rc):
            for hv in range(_HV):
                sl = (r, pl.ds(hv * 16, 16))
                ev_v[sl] = jnp.maximum(xr_v[sl] + ev_v[sl], 0.0)
            return rc
        lax.fori_loop(0, _CHUNK, row_body, 0)

        pltpu.sync_copy(ev_v, aggr.at[dst_v], add=True)
        return carry
    lax.fori_loop(0, _NCHUNK, chunk_body, 0)

    plsc.subcore_barrier()
    pltpu.sync_copy(aggr.at[pl.ds(s * _RPS, _RPS)],
                    out_hbm.at[c, pl.ds(s * _RPS, _RPS)])


_mp_kernel = functools.partial(
    pl.kernel,
    out_type=jax.ShapeDtypeStruct((_NC, _NP, _H), jnp.float32),
    mesh=plsc.VectorSubcoreMesh(core_axis_name="c", subcore_axis_name="s"),
    scratch_types=[
        pltpu.VMEM((_CHUNK,), jnp.int32),
        pltpu.VMEM((_CHUNK,), jnp.int32),
        pltpu.VMEM((_CHUNK, _H), jnp.float32),
        pltpu.VMEM((_CHUNK, _H), jnp.float32),
        pltpu.VMEM_SHARED((_NP, _H), jnp.float32),
        pltpu.SemaphoreType.DMA,
    ],
)(_mp_body)


_UPD_BLK = 2000


def _update_call(x, agg, w1, b1, w2, b2):
    return pl.pallas_call(
        _update_body,
        grid=(_N // _UPD_BLK,),
        in_specs=[
            pl.BlockSpec((_UPD_BLK, _H), lambda i: (i, 0)),
            pl.BlockSpec((_NC, _UPD_BLK, _H), lambda i: (0, i, 0)),

            pl.BlockSpec((_H, _H), lambda i: (0, 0)),
            pl.BlockSpec((1, _H), lambda i: (0, 0)),
            pl.BlockSpec((_H, _H), lambda i: (0, 0)),
            pl.BlockSpec((1, _H), lambda i: (0, 0)),
        ],
        out_specs=pl.BlockSpec((_UPD_BLK, _H), lambda i: (i, 0)),
        out_shape=jax.ShapeDtypeStruct((_N, _H), jnp.float32),
    )(x, agg, w1, b1.reshape(1, _H), w2, b2.reshape(1, _H))


def kernel(s_node_ids, s_edge_index, s_edge_attr, s_batch, depth,
           id_W1, id_b1, id_W2, id_b2, e_W1, e_b1, e_W2, e_b2,
           c1_W1, c1_b1, c1_W2, c1_b2, c2_W1, c2_b1, c2_W2, c2_b2,
           r_W1, r_b1, r_W2, r_b2):
    src = s_edge_index[0].astype(jnp.int32)
    dst = s_edge_index[1].astype(jnp.int32)
    ids2 = s_node_ids.reshape(_N, 1).astype(jnp.int32)

    x = pl.pallas_call(
        _node_mlp_body,
        out_shape=jax.ShapeDtypeStruct((_N, _H), jnp.float32),
    )(ids2, id_W1, id_b1.reshape(1, _H), id_W2, id_b2.reshape(1, _H))

    _EDGE_BLK = 3200
    e = pl.pallas_call(
        _edge_mlp_body,
        grid=(_E // _EDGE_BLK,),
        in_specs=[
            pl.BlockSpec((_EDGE_BLK, 16), lambda i: (i, 0)),
            pl.BlockSpec((16, _H), lambda i: (0, 0)),
            pl.BlockSpec((1, _H), lambda i: (0, 0)),
            pl.BlockSpec((_H, _H), lambda i: (0, 0)),
            pl.BlockSpec((1, _H), lambda i: (0, 0)),
        ],
        out_specs=pl.BlockSpec((_EDGE_BLK, _H), lambda i: (i, 0)),
        out_shape=jax.ShapeDtypeStruct((_E, _H), jnp.float32),
    )(s_edge_attr, e_W1, e_b1.reshape(1, _H), e_W2, e_b2.reshape(1, _H))

    zeros = jnp.zeros((_NP, _H), jnp.float32)

    agg1 = _mp_kernel(x, e, src, dst, zeros)
    h1 = _update_call(x, agg1, c1_W1, c1_b1, c1_W2, c1_b2)

    agg2 = _mp_kernel(h1, e, src, dst, zeros)
    h2 = _update_call(h1, agg2, c2_W1, c2_b1, c2_W2, c2_b2)

    out = pl.pallas_call(
        _pool_head_body,
        out_shape=jax.ShapeDtypeStruct((_B, 1), jnp.float32),
    )(h2, s_batch.reshape(_N, 1).astype(jnp.int32), depth.reshape(_B, 1),
      r_W1[:_H], r_W1[_H:], r_b1.reshape(1, _H), r_W2, r_b2.reshape(1, 1))
    return out[:, 0]
